# baseline (speedup 1.0000x reference)
import functools

import jax
import jax.numpy as jnp
from jax import lax
from jax.experimental import pallas as pl
from jax.experimental.pallas import tpu as pltpu
from jax.experimental.pallas import tpu_sc as plsc

BATCH = 4096
SEQ = 200
D = 64
TOT = BATCH * SEQ
EPS = 1e-6

_info = plsc.get_sparse_core_info()
NC, NS = _info.num_cores, _info.num_subcores
NW = NC * NS
PW = TOT // NW
CH = 128
NCH = PW // CH
POS_WORDS = SEQ * D
U = 16


def _allsum(v, iota):
    for k in (8, 4, 2, 1):
        v = v + v.at[jnp.bitwise_xor(iota, k)].get(mode="promise_in_bounds")
    return v


def _body(tok_hbm, tab_hbm, pos_hbm, gb_hbm, out_hbm,
          idx_v, idx2_v, in_v, out_v, pos_v, gb_v,
          gsem0, gsem1, ssem0, ssem1):
    w = lax.axis_index("s") * NC + lax.axis_index("c")
    base = w * PW

    pltpu.sync_copy(tok_hbm.at[pl.ds(w * NCH, NCH)], idx_v)
    pltpu.sync_copy(pos_hbm.at[pl.ds(0, POS_WORDS)], pos_v)
    pltpu.sync_copy(gb_hbm, gb_v)

    g = [gb_v[pl.ds(i * 16, 16)] for i in range(4)]
    b = [gb_v[pl.ds(D + i * 16, 16)] for i in range(4)]
    iota = lax.iota(jnp.int32, 16)
    gsems = (gsem0, gsem1)
    ssems = (ssem0, ssem1)

    def prep_idx2(c, half):
        for k in range(CH // 16):
            idx2_v[half, pl.ds(k * 16, 16)] = lax.shift_right_logical(
                idx_v[c, pl.ds(k * 16, 16)], 1)

    def issue_gather(c, half):
        del c
        pltpu.async_copy(
            tab_hbm.at[idx2_v.at[half]],
            in_v.at[pl.ds(half * CH, CH)],
            gsems[half],
        )

    def wait_gather(half):
        pltpu.make_async_copy(
            tab_hbm.at[idx2_v.at[half]],
            in_v.at[pl.ds(half * CH, CH)],
            gsems[half],
        ).wait()

    def store_desc(c, half):
        return pltpu.make_async_copy(
            out_v.at[half],
            out_hbm.at[pl.ds((base + c * CH) * D, CH * D)],
            ssems[half],
        )

    def compute(c, half):
        p0 = lax.rem(c * CH, SEQ) * D

        def row_group(i, p):
            tvec = idx_v[c, pl.ds(i * U, 16)]
            offv = (tvec & 1) * D
            for u in range(U):
                j = i * U + u
                r = half * CH + j
                off = offv[u]
                x = [in_v[r, pl.ds(off + i2 * 16, 16)]
                     + pos_v[pl.ds(p + i2 * 16, 16)] for i2 in range(4)]
                s = _allsum((x[0] + x[1]) + (x[2] + x[3]), iota)
                ss = _allsum((x[0] * x[0] + x[1] * x[1])
                             + (x[2] * x[2] + x[3] * x[3]), iota)
                mean = s * (1.0 / D)
                var = ss * (1.0 / D) - mean * mean
                tv = var + EPS
                iv = lax.bitcast_convert_type(tv, jnp.int32)
                iv = 1597463007 - lax.shift_right_logical(iv, 1)
                y = lax.bitcast_convert_type(iv, jnp.float32)
                h = tv * 0.5
                y = y * (1.5 - h * y * y)
                y = y * (1.5 - h * y * y)
                for i2 in range(4):
                    out_v[half, pl.ds(j * D + i2 * 16, 16)] = (
                        (x[i2] - mean) * (y * g[i2]) + b[i2])
                p = p + D
                p = jnp.where(p == POS_WORDS, 0, p)
            return p

        lax.fori_loop(0, CH // U, row_group, p0)

    prep_idx2(0, 0)
    issue_gather(0, 0)
    prep_idx2(1, 1)
    issue_gather(1, 1)

    def pair(i, _):
        for half in range(2):
            c = 2 * i + half
            wait_gather(half)

            @pl.when(i >= 1)
            def _():
                store_desc(c - 2, half).wait()

            compute(c, half)
            store_desc(c, half).start()

            @pl.when(c + 2 < NCH)
            def _():
                prep_idx2(c + 2, half)
                issue_gather(c + 2, half)
        return 0

    lax.fori_loop(0, NCH // 2, pair, 0)
    store_desc(NCH - 2, 0).wait()
    store_desc(NCH - 1, 1).wait()


@jax.jit
def _run(tok2d, table, posflat, gb):
    table128 = table.reshape(-1, 128)
    mesh = plsc.VectorSubcoreMesh(core_axis_name="c", subcore_axis_name="s")
    f = functools.partial(
        pl.kernel,
        mesh=mesh,
        out_type=jax.ShapeDtypeStruct((TOT * D,), jnp.float32),
        scratch_types=[
            pltpu.VMEM((NCH, CH), jnp.int32),
            pltpu.VMEM((2, CH), jnp.int32),
            pltpu.VMEM((2 * CH, 128), jnp.float32),
            pltpu.VMEM((2, CH * D), jnp.float32),
            pltpu.VMEM((POS_WORDS,), jnp.float32),
            pltpu.VMEM((2 * D,), jnp.float32),
            pltpu.SemaphoreType.DMA,
            pltpu.SemaphoreType.DMA,
            pltpu.SemaphoreType.DMA,
            pltpu.SemaphoreType.DMA,
        ],
        compiler_params=pltpu.CompilerParams(use_tc_tiling_on_sc=True),
    )(_body)
    return f(tok2d, table128, posflat, gb)


def kernel(inputs, token_table, pos_table, ln_gamma, ln_beta):
    tok2d = inputs.reshape(TOT // CH, CH).astype(jnp.int32)
    posflat = pos_table.reshape(-1)
    gb = jnp.concatenate([ln_gamma, ln_beta])
    out = _run(tok2d, token_table, posflat, gb)
    return out.reshape(BATCH, SEQ, D)


# speedup vs baseline: 1.5287x; 1.5287x over previous
import functools

import jax
import jax.numpy as jnp
from jax import lax
from jax.experimental import pallas as pl
from jax.experimental.pallas import tpu as pltpu
from jax.experimental.pallas import tpu_sc as plsc

BATCH = 4096
SEQ = 200
D = 64
TOT = BATCH * SEQ
EPS = 1e-6

_info = plsc.get_sparse_core_info()
NC, NS = _info.num_cores, _info.num_subcores
NW = NC * NS
PW = TOT // NW
NCH = BATCH // NW
POS_WORDS = SEQ * D
U = 8


def _allsum(v, iota):
    for k in (8, 4, 2, 1):
        v = v + v.at[jnp.bitwise_xor(iota, k)].get(mode="promise_in_bounds")
    return v


def _body(tok_hbm, tab_hbm, pos_hbm, gb_hbm, out_hbm,
          idx_v, in_v, out_v, pos_v, gb_v,
          gsem0, gsem1, ssem0, ssem1):
    w = lax.axis_index("s") * NC + lax.axis_index("c")

    pltpu.sync_copy(tok_hbm.at[pl.ds(w * PW, PW)], idx_v)
    pltpu.sync_copy(pos_hbm.at[pl.ds(0, SEQ)], pos_v)
    pltpu.sync_copy(gb_hbm, gb_v)

    g = [gb_v[pl.ds(i * 16, 16)] for i in range(4)]
    b = [gb_v[pl.ds(D + i * 16, 16)] for i in range(4)]
    iota = lax.iota(jnp.int32, 16)
    gsems = (gsem0, gsem1)
    ssems = (ssem0, ssem1)

    def gather_descs(c, half):
        return [
            pltpu.make_async_copy(
                tab_hbm.at[idx_v.at[pl.ds(c * SEQ, 128)]],
                in_v.at[half, pl.ds(0, 128)],
                gsems[half],
            ),
            pltpu.make_async_copy(
                tab_hbm.at[idx_v.at[pl.ds(c * SEQ + 128, 72)]],
                in_v.at[half, pl.ds(128, 72)],
                gsems[half],
            ),
        ]

    def store_desc(c, half):
        return pltpu.make_async_copy(
            out_v.at[half],
            out_hbm.at[w * NCH + c],
            ssems[half],
        )

    def compute(half):
        def row_group(i, _):
            for u in range(U):
                j = i * U + u
                x = [in_v[half, j, pl.ds(i2 * 16, 16)]
                     + pos_v[j, pl.ds(i2 * 16, 16)] for i2 in range(4)]
                s = _allsum((x[0] + x[1]) + (x[2] + x[3]), iota)
                ss = _allsum((x[0] * x[0] + x[1] * x[1])
                             + (x[2] * x[2] + x[3] * x[3]), iota)
                mean = s * (1.0 / D)
                var = ss * (1.0 / D) - mean * mean
                tv = var + EPS
                iv = lax.bitcast_convert_type(tv, jnp.int32)
                iv = 1597463007 - lax.shift_right_logical(iv, 1)
                y = lax.bitcast_convert_type(iv, jnp.float32)
                h = tv * 0.5
                y = y * (1.5 - h * y * y)
                y = y * (1.5 - h * y * y)
                for i2 in range(4):
                    out_v[half, j, pl.ds(i2 * 16, 16)] = (
                        (x[i2] - mean) * (y * g[i2]) + b[i2])
            return 0

        lax.fori_loop(0, SEQ // U, row_group, 0)

    for d in gather_descs(0, 0):
        d.start()
    for d in gather_descs(1, 1):
        d.start()

    def pair(i, _):
        for half in range(2):
            c = 2 * i + half
            for d in gather_descs(c, half):
                d.wait()

            @pl.when(i >= 1)
            def _():
                store_desc(c - 2, half).wait()

            compute(half)
            store_desc(c, half).start()

            @pl.when(c + 2 < NCH)
            def _():
                for d in gather_descs(c + 2, half):
                    d.start()
        return 0

    lax.fori_loop(0, NCH // 2, pair, 0)
    store_desc(NCH - 2, 0).wait()
    store_desc(NCH - 1, 1).wait()


@jax.jit
def _run(tok, table, pos, gb):
    mesh = plsc.VectorSubcoreMesh(core_axis_name="c", subcore_axis_name="s")
    f = functools.partial(
        pl.kernel,
        mesh=mesh,
        out_type=jax.ShapeDtypeStruct((BATCH, SEQ, D), jnp.float32),
        scratch_types=[
            pltpu.VMEM((PW,), jnp.int32),
            pltpu.VMEM((2, SEQ, D), jnp.float32),
            pltpu.VMEM((2, SEQ, D), jnp.float32),
            pltpu.VMEM((SEQ, D), jnp.float32),
            pltpu.VMEM((2 * D,), jnp.float32),
            pltpu.SemaphoreType.DMA,
            pltpu.SemaphoreType.DMA,
            pltpu.SemaphoreType.DMA,
            pltpu.SemaphoreType.DMA,
        ],
        compiler_params=pltpu.CompilerParams(use_tc_tiling_on_sc=False),
    )(_body)
    return f(tok, table, pos, gb)


def kernel(inputs, token_table, pos_table, ln_gamma, ln_beta):
    tok = inputs.reshape(-1).astype(jnp.int32)
    gb = jnp.concatenate([ln_gamma, ln_beta])
    return _run(tok, token_table, pos_table, gb)


# speedup vs baseline: 1.5382x; 1.0062x over previous
import functools

import jax
import jax.numpy as jnp
from jax import lax
from jax.experimental import pallas as pl
from jax.experimental.pallas import tpu as pltpu
from jax.experimental.pallas import tpu_sc as plsc

BATCH = 4096
SEQ = 200
D = 64
TOT = BATCH * SEQ
EPS = 1e-6

_info = plsc.get_sparse_core_info()
NC, NS = _info.num_cores, _info.num_subcores
NW = NC * NS
PW = TOT // NW
NCH = BATCH // NW
POS_WORDS = SEQ * D
U = 8


def _allsum(v, iota):
    for k in (8, 4, 2, 1):
        v = v + v.at[jnp.bitwise_xor(iota, k)].get(mode="promise_in_bounds")
    return v


def _body(tok_hbm, tab_hbm, pos_hbm, gb_hbm, out_hbm,
          idx_v, in_v, out_v, pos_v, gb_v,
          gsem0, gsem1, ssem0, ssem1):
    w = lax.axis_index("s") * NC + lax.axis_index("c")

    pltpu.sync_copy(tok_hbm.at[pl.ds(w * PW, PW)], idx_v)
    pltpu.sync_copy(pos_hbm.at[pl.ds(0, SEQ)], pos_v)
    pltpu.sync_copy(gb_hbm, gb_v)

    g = [gb_v[pl.ds(i * 16, 16)] for i in range(4)]
    b = [gb_v[pl.ds(D + i * 16, 16)] for i in range(4)]
    iota = lax.iota(jnp.int32, 16)
    gsems = (gsem0, gsem1)
    ssems = (ssem0, ssem1)

    def gather_descs(c, half):
        return [
            pltpu.make_async_copy(
                tab_hbm.at[idx_v.at[pl.ds(c * SEQ, 128)]],
                in_v.at[half, pl.ds(0, 128)],
                gsems[half],
            ),
            pltpu.make_async_copy(
                tab_hbm.at[idx_v.at[pl.ds(c * SEQ + 128, 72)]],
                in_v.at[half, pl.ds(128, 72)],
                gsems[half],
            ),
        ]

    def store_desc(c, half):
        return pltpu.make_async_copy(
            out_v.at[half],
            out_hbm.at[w * NCH + c],
            ssems[half],
        )

    def compute(half):
        @plsc.parallel_loop(0, SEQ, 1, unroll=U)
        def _(j):
            x = [in_v[half, j, pl.ds(i2 * 16, 16)]
                 + pos_v[j, pl.ds(i2 * 16, 16)] for i2 in range(4)]
            s = _allsum((x[0] + x[1]) + (x[2] + x[3]), iota)
            ss = _allsum((x[0] * x[0] + x[1] * x[1])
                         + (x[2] * x[2] + x[3] * x[3]), iota)
            mean = s * (1.0 / D)
            var = ss * (1.0 / D) - mean * mean
            tv = var + EPS
            iv = lax.bitcast_convert_type(tv, jnp.int32)
            iv = 1597463007 - lax.shift_right_logical(iv, 1)
            y = lax.bitcast_convert_type(iv, jnp.float32)
            h = tv * 0.5
            y = y * (1.5 - h * y * y)
            y = y * (1.5 - h * y * y)
            for i2 in range(4):
                out_v[half, j, pl.ds(i2 * 16, 16)] = (
                    (x[i2] - mean) * (y * g[i2]) + b[i2])

    for d in gather_descs(0, 0):
        d.start()
    for d in gather_descs(1, 1):
        d.start()

    def pair(i, _):
        for half in range(2):
            c = 2 * i + half
            for d in gather_descs(c, half):
                d.wait()

            @pl.when(i >= 1)
            def _():
                store_desc(c - 2, half).wait()

            compute(half)
            store_desc(c, half).start()

            @pl.when(c + 2 < NCH)
            def _():
                for d in gather_descs(c + 2, half):
                    d.start()
        return 0

    lax.fori_loop(0, NCH // 2, pair, 0)
    store_desc(NCH - 2, 0).wait()
    store_desc(NCH - 1, 1).wait()


@jax.jit
def _run(tok, table, pos, gb):
    mesh = plsc.VectorSubcoreMesh(core_axis_name="c", subcore_axis_name="s")
    f = functools.partial(
        pl.kernel,
        mesh=mesh,
        out_type=jax.ShapeDtypeStruct((BATCH, SEQ, D), jnp.float32),
        scratch_types=[
            pltpu.VMEM((PW,), jnp.int32),
            pltpu.VMEM((2, SEQ, D), jnp.float32),
            pltpu.VMEM((2, SEQ, D), jnp.float32),
            pltpu.VMEM((SEQ, D), jnp.float32),
            pltpu.VMEM((2 * D,), jnp.float32),
            pltpu.SemaphoreType.DMA,
            pltpu.SemaphoreType.DMA,
            pltpu.SemaphoreType.DMA,
            pltpu.SemaphoreType.DMA,
        ],
        compiler_params=pltpu.CompilerParams(use_tc_tiling_on_sc=False),
    )(_body)
    return f(tok, table, pos, gb)


def kernel(inputs, token_table, pos_table, ln_gamma, ln_beta):
    tok = inputs.reshape(-1).astype(jnp.int32)
    gb = jnp.concatenate([ln_gamma, ln_beta])
    return _run(tok, token_table, pos_table, gb)
